# baseline (speedup 1.0000x reference)
import functools

import jax
import jax.numpy as jnp
from jax import lax
from jax.experimental import pallas as pl
from jax.experimental.pallas import tpu as pltpu
from jax.experimental.pallas import tpu_sc as plsc

N_NODES = 10000
ACC = 10240
DUMMY = N_NODES
NC, NS = 2, 16
NW = NC * NS
CHUNK = 128
ZROWS = ACC // NS


def _sc_mesh():
    return plsc.VectorSubcoreMesh(core_axis_name="c", subcore_axis_name="s")


_UNTILED = pltpu.CompilerParams(use_tc_tiling_on_sc=False)


def _make_deg_kernel(idx_rows):
    @functools.partial(
        pl.kernel,
        out_type=jax.ShapeDtypeStruct((NC, ACC, 16), jnp.float32),
        mesh=_sc_mesh(),
        compiler_params=_UNTILED,
        scratch_types=[
            pltpu.VMEM((idx_rows, CHUNK), jnp.int32),
            pltpu.VMEM((CHUNK, 16), jnp.float32),
            pltpu.VMEM((ZROWS, 16), jnp.float32),
            pltpu.VMEM_SHARED((ACC, 16), jnp.float32),
        ],
    )
    def deg_kernel(edges_hbm, out_hbm, dst_v, ones_v, zbuf_v, acc_sh):
        cid = lax.axis_index("c")
        sid = lax.axis_index("s")
        wid = cid * NS + sid

        zero16 = jnp.zeros((16,), jnp.float32)
        one16 = jnp.ones((16,), jnp.float32)

        def _fill_z(i, carry):
            zbuf_v[i] = zero16
            return carry

        lax.fori_loop(0, ZROWS, _fill_z, None)

        def _fill_o(i, carry):
            ones_v[i] = one16
            return carry

        lax.fori_loop(0, CHUNK, _fill_o, None)

        pltpu.sync_copy(zbuf_v, acc_sh.at[pl.ds(sid * ZROWS, ZROWS)])
        plsc.subcore_barrier()

        pltpu.sync_copy(edges_hbm.at[1, pl.ds(wid * idx_rows, idx_rows)], dst_v)

        def _chunk(j, carry):
            pltpu.sync_copy(ones_v, acc_sh.at[dst_v.at[j]], add=True)
            return carry

        lax.fori_loop(0, idx_rows, _chunk, None)
        plsc.subcore_barrier()

        pltpu.sync_copy(
            acc_sh.at[pl.ds(sid * ZROWS, ZROWS)],
            out_hbm.at[cid, pl.ds(sid * ZROWS, ZROWS)],
        )

    return deg_kernel


def _make_edge_scatter_kernel(rows_c0, rows_c1, d, grp, dtype=jnp.float32,
                              stage_rows=0):
    assert rows_c0 % (2 * grp) == 0 and rows_c1 % (2 * grp) == 0
    buf_rows = 2 * grp * CHUNK
    idx_rows = max(rows_c0, rows_c1)
    vec = 32 if dtype == jnp.bfloat16 else 16
    stage_scratch = (
        [pltpu.VMEM_SHARED((stage_rows, d), dtype)] if stage_rows else [])

    @functools.partial(
        pl.kernel,
        out_type=jax.ShapeDtypeStruct((NC, ACC, d), dtype),
        mesh=_sc_mesh(),
        compiler_params=_UNTILED if (d % 128 != 0 or dtype != jnp.float32) else None,
        scratch_types=[
            pltpu.VMEM((idx_rows, CHUNK), jnp.int32),
            pltpu.VMEM((idx_rows, CHUNK), jnp.int32),
            pltpu.VMEM((buf_rows, d), dtype),
            pltpu.VMEM_SHARED((ACC, d), dtype),
            pltpu.SemaphoreType.DMA,
            pltpu.SemaphoreType.DMA,
        ] + stage_scratch,
    )
    def scatter_kernel(ys_hbm, edges_hbm, out_hbm,
                       src_v, dst_v, rows_v, acc_sh, sem_a, sem_b,
                       *maybe_stage):
        cid = lax.axis_index("c")
        sid = lax.axis_index("s")
        if stage_rows:
            ys_sh = maybe_stage[0]
            per = stage_rows // NS
            pltpu.sync_copy(ys_hbm.at[pl.ds(sid * per, per)],
                            ys_sh.at[pl.ds(sid * per, per)])
            ys_src = ys_sh
        else:
            ys_src = ys_hbm

        zerov = jnp.zeros((vec,), dtype)
        nvec = d // vec

        def _fill_z(i, carry):
            for v in range(nvec):
                rows_v[i, pl.ds(v * vec, vec)] = zerov
            return carry

        lax.fori_loop(0, buf_rows, _fill_z, None)

        off = 0
        while off < ZROWS:
            step = min(buf_rows, ZROWS - off)
            pltpu.sync_copy(
                rows_v.at[pl.ds(0, step)],
                acc_sh.at[pl.ds(sid * ZROWS + off, step)],
            )
            off += step
        plsc.subcore_barrier()

        def _gather_group(gi, buf, sem):
            base = gi * grp
            for t in range(grp):
                pltpu.async_copy(
                    ys_src.at[src_v.at[base + t]],
                    rows_v.at[pl.ds((buf * grp + t) * CHUNK, CHUNK)],
                    sem,
                )

        def _wait_group(buf, sem):
            for t in range(grp):
                pltpu.make_async_copy(
                    ys_src.at[src_v.at[0]],
                    rows_v.at[pl.ds((buf * grp + t) * CHUNK, CHUNK)],
                    sem,
                ).wait()

        def _scatter_group(gi, buf):
            base = gi * grp
            for t in range(grp):
                pltpu.sync_copy(
                    rows_v.at[pl.ds((buf * grp + t) * CHUNK, CHUNK)],
                    acc_sh.at[dst_v.at[base + t]],
                    add=True,
                )

        def _run(row0, nrows):
            if nrows == 0:
                return
            pltpu.sync_copy(edges_hbm.at[0, pl.ds(row0, nrows)],
                            src_v.at[pl.ds(0, nrows)])
            pltpu.sync_copy(edges_hbm.at[1, pl.ds(row0, nrows)],
                            dst_v.at[pl.ds(0, nrows)])
            half = nrows // (2 * grp)
            _gather_group(0, 0, sem_a)

            def _pipe(q, carry):
                _gather_group(2 * q + 1, 1, sem_b)
                _wait_group(0, sem_a)
                _scatter_group(2 * q, 0)

                @pl.when(q < half - 1)
                def _prefetch():
                    _gather_group(2 * q + 2, 0, sem_a)

                _wait_group(1, sem_b)
                _scatter_group(2 * q + 1, 1)
                return carry

            lax.fori_loop(0, half, _pipe, None)

        @pl.when(cid == 0)
        def _run_c0():
            _run(sid * rows_c0, rows_c0)

        @pl.when(cid == 1)
        def _run_c1():
            _run(NS * rows_c0 + sid * rows_c1, rows_c1)

        plsc.subcore_barrier()

        pltpu.sync_copy(
            acc_sh.at[pl.ds(sid * ZROWS, ZROWS)],
            out_hbm.at[cid, pl.ds(sid * ZROWS, ZROWS)],
        )

    return scatter_kernel


RB = 2000


def _dis_block(degp_ref):
    deg = degp_ref[0, :, 0:1] + degp_ref[1, :, 0:1] + 1.0
    return lax.rsqrt(deg)


def _tc1_body(x_ref, w1_ref, degp_ref, y1_ref, ys1_ref):
    y = jnp.dot(x_ref[...], w1_ref[...], preferred_element_type=jnp.float32)
    dis = _dis_block(degp_ref)
    y1_ref[...] = y
    ys1_ref[...] = (y * dis).astype(jnp.bfloat16)


def _tc2_body(z1p_ref, y1_ref, degp_ref, b1_ref, w2_ref, y2_ref, ys2_ref):
    dis = _dis_block(degp_ref)
    z = z1p_ref[0].astype(jnp.float32) + z1p_ref[1].astype(jnp.float32)
    h = jnp.maximum(dis * z + (dis * dis) * y1_ref[...] + b1_ref[...], 0.0)
    y2 = jnp.dot(h, w2_ref[...], preferred_element_type=jnp.float32)
    y2_ref[...] = y2
    ys2_ref[...] = y2 * dis


def _tc3_body(z2p_ref, y2_ref, degp_ref, b2_ref, out_ref):
    dis = _dis_block(degp_ref)
    z = z2p_ref[0] + z2p_ref[1]
    o = dis * z + (dis * dis) * y2_ref[...] + b2_ref[...]
    m = jnp.max(o, axis=1, keepdims=True)
    e = o - m
    lse = jnp.log(jnp.sum(jnp.exp(e), axis=1, keepdims=True))
    out_ref[...] = e - lse


def _row_spec(d):
    return pl.BlockSpec((RB, d), lambda i: (i, 0))


def _part_spec(d):
    return pl.BlockSpec((NC, RB, d), lambda i: (0, i, 0))


def _full_spec(shape):
    nd = len(shape)
    return pl.BlockSpec(shape, lambda i: (0,) * nd)


def kernel(x, edge_index, W1, b1, W2, b2):
    n, in_dim = x.shape
    hid = W1.shape[1]
    out_dim = W2.shape[1]
    e = edge_index.shape[1]

    idx_rows_total = -(-e // (NW * CHUNK)) * NW
    e_pad = idx_rows_total * CHUNK
    idx_rows = idx_rows_total // NW
    pad_cols = jnp.broadcast_to(
        jnp.array([[0], [DUMMY]], jnp.int32), (2, e_pad - e))
    edges3 = jnp.concatenate(
        [edge_index.astype(jnp.int32), pad_cols], axis=1,
    ).reshape(2, idx_rows_total, CHUNK)
    b1r = b1.reshape(1, hid)
    b2r = b2.reshape(1, out_dim)

    grid = n // RB

    degp = _make_deg_kernel(idx_rows)(edges3)

    y1, ys1 = pl.pallas_call(
        _tc1_body,
        grid=(grid,),
        in_specs=[
            _row_spec(in_dim),
            _full_spec((in_dim, hid)),
            _part_spec(16),
        ],
        out_specs=[_row_spec(hid), _row_spec(hid)],
        out_shape=[
            jax.ShapeDtypeStruct((n, hid), jnp.float32),
            jax.ShapeDtypeStruct((n, hid), jnp.bfloat16),
        ],
    )(x, W1, degp)

    rows_c0 = 2 * idx_rows - ((2 * idx_rows * 3) // 10) // 8 * 8
    rows_c1 = 2 * idx_rows - rows_c0
    z1p = _make_edge_scatter_kernel(rows_c0, rows_c1, hid, 2,
                                    jnp.bfloat16)(ys1, edges3)

    y2, ys2 = pl.pallas_call(
        _tc2_body,
        grid=(grid,),
        in_specs=[
            _part_spec(hid),
            _row_spec(hid),
            _part_spec(16),
            _full_spec((1, hid)),
            _full_spec((hid, out_dim)),
        ],
        out_specs=[_row_spec(out_dim), _row_spec(out_dim)],
        out_shape=[
            jax.ShapeDtypeStruct((n, out_dim), jnp.float32),
            jax.ShapeDtypeStruct((n, out_dim), jnp.float32),
        ],
    )(z1p, y1, degp, b1r, W2)

    z2p = _make_edge_scatter_kernel(rows_c0, rows_c1, out_dim, 2,
                                    stage_rows=n)(ys2, edges3)

    out = pl.pallas_call(
        _tc3_body,
        grid=(grid,),
        in_specs=[
            _part_spec(out_dim),
            _row_spec(out_dim),
            _part_spec(16),
            _full_spec((1, out_dim)),
        ],
        out_specs=_row_spec(out_dim),
        out_shape=jax.ShapeDtypeStruct((n, out_dim), jnp.float32),
    )(z2p, y2, degp, b2r)

    return out


# speedup vs baseline: 1.0253x; 1.0253x over previous
import functools

import jax
import jax.numpy as jnp
from jax import lax
from jax.experimental import pallas as pl
from jax.experimental.pallas import tpu as pltpu
from jax.experimental.pallas import tpu_sc as plsc

N_NODES = 10000
ACC = 10240
DUMMY = N_NODES
NC, NS = 2, 16
NW = NC * NS
CHUNK = 128
ZROWS = ACC // NS


def _sc_mesh():
    return plsc.VectorSubcoreMesh(core_axis_name="c", subcore_axis_name="s")


_UNTILED = pltpu.CompilerParams(use_tc_tiling_on_sc=False)


def _make_deg_kernel(idx_rows):
    @functools.partial(
        pl.kernel,
        out_type=jax.ShapeDtypeStruct((NC, ACC, 16), jnp.float32),
        mesh=_sc_mesh(),
        compiler_params=_UNTILED,
        scratch_types=[
            pltpu.VMEM((idx_rows, CHUNK), jnp.int32),
            pltpu.VMEM((CHUNK, 16), jnp.float32),
            pltpu.VMEM((ZROWS, 16), jnp.float32),
            pltpu.VMEM_SHARED((ACC, 16), jnp.float32),
        ],
    )
    def deg_kernel(dst_hbm, out_hbm, dst_v, ones_v, zbuf_v, acc_sh):
        cid = lax.axis_index("c")
        sid = lax.axis_index("s")
        wid = cid * NS + sid

        zero16 = jnp.zeros((16,), jnp.float32)
        one16 = jnp.ones((16,), jnp.float32)

        def _fill_z(i, carry):
            zbuf_v[i] = zero16
            return carry

        lax.fori_loop(0, ZROWS, _fill_z, None)

        def _fill_o(i, carry):
            ones_v[i] = one16
            return carry

        lax.fori_loop(0, CHUNK, _fill_o, None)

        pltpu.sync_copy(zbuf_v, acc_sh.at[pl.ds(sid * ZROWS, ZROWS)])
        plsc.subcore_barrier()

        pltpu.sync_copy(dst_hbm.at[pl.ds(wid * idx_rows, idx_rows)], dst_v)

        def _chunk(j, carry):
            pltpu.sync_copy(ones_v, acc_sh.at[dst_v.at[j]], add=True)
            return carry

        lax.fori_loop(0, idx_rows, _chunk, None)
        plsc.subcore_barrier()

        pltpu.sync_copy(
            acc_sh.at[pl.ds(sid * ZROWS, ZROWS)],
            out_hbm.at[cid, pl.ds(sid * ZROWS, ZROWS)],
        )

    return deg_kernel


def _make_edge_scatter_kernel(rows_c0, rows_c1, d, grp, dtype=jnp.float32,
                              stage_rows=0):
    assert rows_c0 % (2 * grp) == 0 and rows_c1 % (2 * grp) == 0
    buf_rows = 2 * grp * CHUNK
    idx_rows = max(rows_c0, rows_c1)
    vec = 32 if dtype == jnp.bfloat16 else 16
    stage_scratch = (
        [pltpu.VMEM_SHARED((stage_rows, d), dtype)] if stage_rows else [])

    @functools.partial(
        pl.kernel,
        out_type=jax.ShapeDtypeStruct((NC, ACC, d), dtype),
        mesh=_sc_mesh(),
        compiler_params=_UNTILED if (d % 128 != 0 or dtype != jnp.float32) else None,
        scratch_types=[
            pltpu.VMEM((idx_rows, CHUNK), jnp.int32),
            pltpu.VMEM((idx_rows, CHUNK), jnp.int32),
            pltpu.VMEM((buf_rows, d), dtype),
            pltpu.VMEM_SHARED((ACC, d), dtype),
            pltpu.SemaphoreType.DMA,
            pltpu.SemaphoreType.DMA,
        ] + stage_scratch,
    )
    def scatter_kernel(ys_hbm, src_hbm, dst_hbm, out_hbm,
                       src_v, dst_v, rows_v, acc_sh, sem_a, sem_b,
                       *maybe_stage):
        cid = lax.axis_index("c")
        sid = lax.axis_index("s")
        if stage_rows:
            ys_sh = maybe_stage[0]
            per = stage_rows // NS
            pltpu.sync_copy(ys_hbm.at[pl.ds(sid * per, per)],
                            ys_sh.at[pl.ds(sid * per, per)])
            ys_src = ys_sh
        else:
            ys_src = ys_hbm

        zerov = jnp.zeros((vec,), dtype)
        nvec = d // vec

        def _fill_z(i, carry):
            for v in range(nvec):
                rows_v[i, pl.ds(v * vec, vec)] = zerov
            return carry

        lax.fori_loop(0, buf_rows, _fill_z, None)

        off = 0
        while off < ZROWS:
            step = min(buf_rows, ZROWS - off)
            pltpu.sync_copy(
                rows_v.at[pl.ds(0, step)],
                acc_sh.at[pl.ds(sid * ZROWS + off, step)],
            )
            off += step
        plsc.subcore_barrier()

        def _gather_group(gi, buf, sem):
            base = gi * grp
            for t in range(grp):
                pltpu.async_copy(
                    ys_src.at[src_v.at[base + t]],
                    rows_v.at[pl.ds((buf * grp + t) * CHUNK, CHUNK)],
                    sem,
                )

        def _wait_group(buf, sem):
            for t in range(grp):
                pltpu.make_async_copy(
                    ys_src.at[src_v.at[0]],
                    rows_v.at[pl.ds((buf * grp + t) * CHUNK, CHUNK)],
                    sem,
                ).wait()

        def _scatter_group(gi, buf):
            base = gi * grp
            for t in range(grp):
                pltpu.sync_copy(
                    rows_v.at[pl.ds((buf * grp + t) * CHUNK, CHUNK)],
                    acc_sh.at[dst_v.at[base + t]],
                    add=True,
                )

        def _run(row0, nrows):
            if nrows == 0:
                return
            pltpu.sync_copy(src_hbm.at[pl.ds(row0, nrows)],
                            src_v.at[pl.ds(0, nrows)])
            pltpu.sync_copy(dst_hbm.at[pl.ds(row0, nrows)],
                            dst_v.at[pl.ds(0, nrows)])
            half = nrows // (2 * grp)
            _gather_group(0, 0, sem_a)

            def _pipe(q, carry):
                _gather_group(2 * q + 1, 1, sem_b)
                _wait_group(0, sem_a)
                _scatter_group(2 * q, 0)

                @pl.when(q < half - 1)
                def _prefetch():
                    _gather_group(2 * q + 2, 0, sem_a)

                _wait_group(1, sem_b)
                _scatter_group(2 * q + 1, 1)
                return carry

            lax.fori_loop(0, half, _pipe, None)

        @pl.when(cid == 0)
        def _run_c0():
            _run(sid * rows_c0, rows_c0)

        @pl.when(cid == 1)
        def _run_c1():
            _run(NS * rows_c0 + sid * rows_c1, rows_c1)

        plsc.subcore_barrier()

        pltpu.sync_copy(
            acc_sh.at[pl.ds(sid * ZROWS, ZROWS)],
            out_hbm.at[cid, pl.ds(sid * ZROWS, ZROWS)],
        )

    return scatter_kernel


RB = 2000


def _dis_block(degp_ref):
    deg = degp_ref[0, :, 0:1] + degp_ref[1, :, 0:1] + 1.0
    return lax.rsqrt(deg)


def _tc1_body(x_ref, w1_ref, degp_ref, y1_ref, ys1_ref):
    y = jnp.dot(x_ref[...], w1_ref[...], preferred_element_type=jnp.float32)
    dis = _dis_block(degp_ref)
    y1_ref[...] = y
    ys1_ref[...] = (y * dis).astype(jnp.bfloat16)


def _tc2_body(z1p_ref, y1_ref, degp_ref, b1_ref, w2_ref, y2_ref, ys2_ref):
    dis = _dis_block(degp_ref)
    z = z1p_ref[0].astype(jnp.float32) + z1p_ref[1].astype(jnp.float32)
    h = jnp.maximum(dis * z + (dis * dis) * y1_ref[...] + b1_ref[...], 0.0)
    y2 = jnp.dot(h, w2_ref[...], preferred_element_type=jnp.float32)
    y2_ref[...] = y2
    ys2_ref[...] = y2 * dis


def _tc3_body(z2p_ref, y2_ref, degp_ref, b2_ref, out_ref):
    dis = _dis_block(degp_ref)
    z = z2p_ref[0] + z2p_ref[1]
    o = dis * z + (dis * dis) * y2_ref[...] + b2_ref[...]
    m = jnp.max(o, axis=1, keepdims=True)
    e = o - m
    lse = jnp.log(jnp.sum(jnp.exp(e), axis=1, keepdims=True))
    out_ref[...] = e - lse


def _row_spec(d):
    return pl.BlockSpec((RB, d), lambda i: (i, 0))


def _part_spec(d):
    return pl.BlockSpec((NC, RB, d), lambda i: (0, i, 0))


def _full_spec(shape):
    nd = len(shape)
    return pl.BlockSpec(shape, lambda i: (0,) * nd)


def kernel(x, edge_index, W1, b1, W2, b2):
    n, in_dim = x.shape
    hid = W1.shape[1]
    out_dim = W2.shape[1]
    e = edge_index.shape[1]

    idx_rows_total = -(-e // (NW * CHUNK)) * NW
    e_pad = idx_rows_total * CHUNK
    idx_rows = idx_rows_total // NW
    src = jnp.concatenate(
        [edge_index[0].astype(jnp.int32),
         jnp.zeros((e_pad - e,), jnp.int32)]).reshape(idx_rows_total, CHUNK)
    dst = jnp.concatenate(
        [edge_index[1].astype(jnp.int32),
         jnp.full((e_pad - e,), DUMMY, jnp.int32)]).reshape(idx_rows_total, CHUNK)
    b1r = b1.reshape(1, hid)
    b2r = b2.reshape(1, out_dim)

    grid = n // RB

    degp = _make_deg_kernel(idx_rows)(dst)

    y1, ys1 = pl.pallas_call(
        _tc1_body,
        grid=(grid,),
        in_specs=[
            _row_spec(in_dim),
            _full_spec((in_dim, hid)),
            _part_spec(16),
        ],
        out_specs=[_row_spec(hid), _row_spec(hid)],
        out_shape=[
            jax.ShapeDtypeStruct((n, hid), jnp.float32),
            jax.ShapeDtypeStruct((n, hid), jnp.bfloat16),
        ],
    )(x, W1, degp)

    rows_c0 = 2 * idx_rows - ((2 * idx_rows * 3) // 10) // 8 * 8
    rows_c1 = 2 * idx_rows - rows_c0
    z1p = _make_edge_scatter_kernel(rows_c0, rows_c1, hid, 2,
                                    jnp.bfloat16)(ys1, src, dst)

    y2, ys2 = pl.pallas_call(
        _tc2_body,
        grid=(grid,),
        in_specs=[
            _part_spec(hid),
            _row_spec(hid),
            _part_spec(16),
            _full_spec((1, hid)),
            _full_spec((hid, out_dim)),
        ],
        out_specs=[_row_spec(out_dim), _row_spec(out_dim)],
        out_shape=[
            jax.ShapeDtypeStruct((n, out_dim), jnp.float32),
            jax.ShapeDtypeStruct((n, out_dim), jnp.float32),
        ],
    )(z1p, y1, degp, b1r, W2)

    z2p = _make_edge_scatter_kernel(rows_c0, rows_c1, out_dim, 2,
                                    stage_rows=n)(ys2, src, dst)

    out = pl.pallas_call(
        _tc3_body,
        grid=(grid,),
        in_specs=[
            _part_spec(out_dim),
            _row_spec(out_dim),
            _part_spec(16),
            _full_spec((1, out_dim)),
        ],
        out_specs=_row_spec(out_dim),
        out_shape=jax.ShapeDtypeStruct((n, out_dim), jnp.float32),
    )(z2p, y2, degp, b2r)

    return out
